# baseline (device time: 13378 ns/iter reference)
import jax
import jax.numpy as jnp
from jax import lax
from jax.experimental import pallas as pl
from jax.experimental.pallas import tpu as pltpu

N_Z = 4
N_CHUNKS = 4


def kernel(x, pi):
    _, m, n = x.shape
    rows = m // N_CHUNKS

    def body(
        x_ref,
        pi_ref,
        out_ref,
        q_send,
        q_recv,
        sc_send,
        sc_recv,
        q_send_sems,
        q_recv_sems,
        sc_send_sem,
        sc_recv_sem,
    ):
        my_x = lax.axis_index("x")
        my_y = lax.axis_index("y")
        my_z = lax.axis_index("z")
        dst_z = pi_ref[my_z]
        src_z = jnp.int32(0)
        for s in range(N_Z):
            src_z = jnp.where(pi_ref[s] == my_z, jnp.int32(s), src_z)

        barrier_sem = pltpu.get_barrier_semaphore()
        pl.semaphore_signal(
            barrier_sem,
            inc=1,
            device_id=(my_x, my_y, src_z),
            device_id_type=pl.DeviceIdType.MESH,
        )

        amax = jnp.maximum(jnp.max(jnp.abs(x_ref[0])), 1e-30)
        inv = 127.0 / amax
        sc_send[0, :] = jnp.full((n,), amax / 127.0, dtype=jnp.float32)

        def quantize(c):
            xc = x_ref[0, pl.ds(c * rows, rows)]
            q_send[pl.ds(c * rows, rows)] = jnp.round(xc * inv).astype(
                jnp.int8
            )

        quantize(0)
        pl.semaphore_wait(barrier_sem, 1)

        sc_rdma = pltpu.make_async_remote_copy(
            src_ref=sc_send,
            dst_ref=sc_recv,
            send_sem=sc_send_sem,
            recv_sem=sc_recv_sem,
            device_id=(my_x, my_y, dst_z),
            device_id_type=pl.DeviceIdType.MESH,
        )
        sc_rdma.start()

        rdmas = []
        for c in range(N_CHUNKS):
            q_rdma = pltpu.make_async_remote_copy(
                src_ref=q_send.at[pl.ds(c * rows, rows)],
                dst_ref=q_recv.at[pl.ds(c * rows, rows)],
                send_sem=q_send_sems.at[c],
                recv_sem=q_recv_sems.at[c],
                device_id=(my_x, my_y, dst_z),
                device_id_type=pl.DeviceIdType.MESH,
            )
            q_rdma.start()
            rdmas.append(q_rdma)
            if c + 1 < N_CHUNKS:
                quantize(c + 1)

        sc_rdma.wait_recv()
        scale = sc_recv[0, 0].astype(jnp.bfloat16)
        for c, q_rdma in enumerate(rdmas):
            q_rdma.wait_recv()
            qc = q_recv[pl.ds(c * rows, rows)].astype(jnp.bfloat16)
            out_ref[0, pl.ds(c * rows, rows)] = qc * scale
        sc_rdma.wait_send()
        for q_rdma in rdmas:
            q_rdma.wait_send()

    return pl.pallas_call(
        body,
        out_shape=jax.ShapeDtypeStruct((1, m, n), jnp.bfloat16),
        in_specs=[
            pl.BlockSpec(memory_space=pltpu.VMEM),
            pl.BlockSpec(memory_space=pltpu.SMEM),
        ],
        out_specs=pl.BlockSpec(memory_space=pltpu.VMEM),
        scratch_shapes=[
            pltpu.VMEM((m, n), jnp.int8),
            pltpu.VMEM((m, n), jnp.int8),
            pltpu.VMEM((1, n), jnp.float32),
            pltpu.VMEM((1, n), jnp.float32),
            pltpu.SemaphoreType.DMA((N_CHUNKS,)),
            pltpu.SemaphoreType.DMA((N_CHUNKS,)),
            pltpu.SemaphoreType.DMA,
            pltpu.SemaphoreType.DMA,
        ],
        compiler_params=pltpu.CompilerParams(collective_id=0),
    )(x, pi)
